# baseline (speedup 1.0000x reference)
import functools
import jax
import jax.numpy as jnp
from jax import lax
from jax.experimental import pallas as pl
from jax.experimental.pallas import tpu as pltpu
from jax.experimental.pallas import tpu_sc as plsc

DIM = 64
NC = 2
NS = 16
NW = NC * NS
NBUF = 2


def _make_gather(total, per_w, chunk, n_chunks):
  mesh = plsc.VectorSubcoreMesh(core_axis_name="c", subcore_axis_name="s")

  @functools.partial(
      pl.kernel,
      mesh=mesh,
      out_type=jax.ShapeDtypeStruct((total, DIM), jnp.float32),
      scratch_types=[
          pltpu.VMEM((per_w,), jnp.int32),
          pltpu.VMEM((chunk, DIM), jnp.float32),
          pltpu.VMEM((chunk, DIM), jnp.float32),
          pltpu.SemaphoreType.DMA,
          pltpu.SemaphoreType.DMA,
      ],
      compiler_params=pltpu.CompilerParams(use_tc_tiling_on_sc=False),
  )
  def body(idx_hbm, table_hbm, out_hbm, idx_v, buf0, buf1, sem0, sem1):
    wid = lax.axis_index("s") * NC + lax.axis_index("c")
    base = wid * per_w
    pltpu.sync_copy(idx_hbm.at[pl.ds(base, per_w)], idx_v)
    bufs = (buf0, buf1)
    sems = (sem0, sem1)
    for b in range(NBUF):
      pltpu.async_copy(
          table_hbm.at[idx_v.at[pl.ds(b * chunk, chunk)]], bufs[b], sems[b])

    @pl.loop(0, n_chunks, step=NBUF)
    def _(c0):
      for b in range(NBUF):
        c = c0 + b
        pltpu.make_async_copy(
            table_hbm.at[pl.ds(0, chunk)], bufs[b], sems[b]).wait()
        pltpu.sync_copy(bufs[b], out_hbm.at[pl.ds(base + c * chunk, chunk)])

        @pl.when(c + NBUF < n_chunks)
        def _():
          pltpu.async_copy(
              table_hbm.at[idx_v.at[pl.ds((c + NBUF) * chunk, chunk)]],
              bufs[b], sems[b])

  return body


def kernel(input_ids, weight):
  batch, fields = input_ids.shape
  total = batch * fields
  per_w = total // NW
  chunk = 512
  n_chunks = per_w // chunk
  idx = input_ids.reshape(total).astype(jnp.int32)
  out = _make_gather(total, per_w, chunk, n_chunks)(idx, weight)
  return out.reshape(batch, fields, DIM)


# speedup vs baseline: 1.0003x; 1.0003x over previous
import functools
import jax
import jax.numpy as jnp
from jax import lax
from jax.experimental import pallas as pl
from jax.experimental.pallas import tpu as pltpu
from jax.experimental.pallas import tpu_sc as plsc

DIM = 64
NC = 2
NS = 16
NW = NC * NS
NBUF = 4


def _make_gather(total, per_w, chunk, n_chunks):
  mesh = plsc.VectorSubcoreMesh(core_axis_name="c", subcore_axis_name="s")

  @functools.partial(
      pl.kernel,
      mesh=mesh,
      out_type=jax.ShapeDtypeStruct((total, DIM), jnp.float32),
      scratch_types=[
          pltpu.VMEM((per_w,), jnp.int32),
          [pltpu.VMEM((chunk, DIM), jnp.float32) for _ in range(NBUF)],
          [pltpu.SemaphoreType.DMA for _ in range(NBUF)],
          [pltpu.SemaphoreType.DMA for _ in range(NBUF)],
      ],
      compiler_params=pltpu.CompilerParams(use_tc_tiling_on_sc=False),
  )
  def body(idx_hbm, table_hbm, out_hbm, idx_v, bufs, gsems, wsems):
    wid = lax.axis_index("s") * NC + lax.axis_index("c")
    base = wid * per_w
    pltpu.sync_copy(idx_hbm.at[pl.ds(base, per_w)], idx_v)
    for b in range(NBUF - 1):
      pltpu.async_copy(
          table_hbm.at[idx_v.at[pl.ds(b * chunk, chunk)]], bufs[b], gsems[b])

    @pl.loop(0, n_chunks, step=NBUF)
    def _(c0):
      for b in range(NBUF):
        c = c0 + b
        g = c + NBUF - 1
        bg = (b + NBUF - 1) % NBUF

        @pl.when(c >= 1)
        def _():
          pltpu.make_async_copy(
              bufs[bg], out_hbm.at[pl.ds(base, chunk)], wsems[bg]).wait()

        @pl.when(g < n_chunks)
        def _():
          pltpu.async_copy(
              table_hbm.at[idx_v.at[pl.ds(g * chunk, chunk)]],
              bufs[bg], gsems[bg])

        pltpu.make_async_copy(
            table_hbm.at[pl.ds(0, chunk)], bufs[b], gsems[b]).wait()
        pltpu.async_copy(
            bufs[b], out_hbm.at[pl.ds(base + c * chunk, chunk)], wsems[b])

    bl = (n_chunks - 1) % NBUF
    pltpu.make_async_copy(
        bufs[bl], out_hbm.at[pl.ds(base, chunk)], wsems[bl]).wait()

  return body


def kernel(input_ids, weight):
  batch, fields = input_ids.shape
  total = batch * fields
  per_w = total // NW
  chunk = 416
  n_chunks = per_w // chunk
  idx = input_ids.reshape(total).astype(jnp.int32)
  out = _make_gather(total, per_w, chunk, n_chunks)(idx, weight)
  return out.reshape(batch, fields, DIM)


# speedup vs baseline: 1.0419x; 1.0416x over previous
import functools
import jax
import jax.numpy as jnp
from jax import lax
from jax.experimental import pallas as pl
from jax.experimental.pallas import tpu as pltpu
from jax.experimental.pallas import tpu_sc as plsc

DIM = 64
NC = 2
NS = 16
NW = NC * NS
NBUF = 4


def _make_gather(total, per_w, chunk, n_chunks):
  mesh = plsc.VectorSubcoreMesh(core_axis_name="c", subcore_axis_name="s")

  @functools.partial(
      pl.kernel,
      mesh=mesh,
      out_type=jax.ShapeDtypeStruct((total, DIM), jnp.float32),
      scratch_types=[
          pltpu.VMEM((per_w,), jnp.int32),
          [pltpu.VMEM((chunk, DIM), jnp.float32) for _ in range(NBUF)],
          [pltpu.SemaphoreType.DMA for _ in range(NBUF)],
          [pltpu.SemaphoreType.DMA for _ in range(NBUF)],
      ],
      compiler_params=pltpu.CompilerParams(use_tc_tiling_on_sc=False),
  )
  def body(idx_hbm, table_hbm, out_hbm, idx_v, bufs, gsems, wsems):
    wid = lax.axis_index("s") * NC + lax.axis_index("c")
    base = wid * per_w
    pltpu.sync_copy(idx_hbm.at[pl.ds(base, per_w)], idx_v)
    for b in range(NBUF - 1):
      pltpu.async_copy(
          table_hbm.at[idx_v.at[pl.ds(b * chunk, chunk)]], bufs[b], gsems[b])

    @pl.loop(0, n_chunks, step=NBUF)
    def _(c0):
      for b in range(NBUF):
        c = c0 + b
        g = c + NBUF - 1
        bg = (b + NBUF - 1) % NBUF

        @pl.when(c >= 1)
        def _():
          pltpu.make_async_copy(
              bufs[bg], out_hbm.at[pl.ds(base, chunk)], wsems[bg]).wait()

        @pl.when(g < n_chunks)
        def _():
          pltpu.async_copy(
              table_hbm.at[idx_v.at[pl.ds(g * chunk, chunk)]],
              bufs[bg], gsems[bg])

        pltpu.make_async_copy(
            table_hbm.at[pl.ds(0, chunk)], bufs[b], gsems[b]).wait()
        pltpu.async_copy(
            bufs[b], out_hbm.at[pl.ds(base + c * chunk, chunk)], wsems[b])

    bl = (n_chunks - 1) % NBUF
    pltpu.make_async_copy(
        bufs[bl], out_hbm.at[pl.ds(base, chunk)], wsems[bl]).wait()

  return body


def kernel(input_ids, weight):
  batch, fields = input_ids.shape
  total = batch * fields
  per_w = total // NW
  chunk = 416
  n_chunks = per_w // chunk
  idx = input_ids.T.reshape(total).astype(jnp.int32)
  out = _make_gather(total, per_w, chunk, n_chunks)(idx, weight)
  return out.reshape(fields, batch, DIM).transpose(1, 0, 2)
